# baseline (speedup 1.0000x reference)
import dataclasses

import jax
import jax.numpy as jnp
from jax import lax
from jax.experimental import pallas as pl
from jax.experimental.pallas import tpu as pltpu
from jax.experimental.pallas import tpu_sc as plsc

NUM_IDX = 16384
EMB = 64
NUM_CORES = 2
B_PER_W = NUM_IDX // NUM_CORES
CHUNK = 512
NUM_CHUNKS = B_PER_W // CHUNK


def kernel(publisher_idx, embedding_publisher):
    idx = publisher_idx.astype(jnp.int32)
    mesh = plsc.ScalarSubcoreMesh(axis_name="c", num_cores=NUM_CORES)
    cp = pltpu.CompilerParams()
    if "needs_layout_passes" in pltpu.CompilerParams.__dataclass_fields__:
        cp = dataclasses.replace(cp, needs_layout_passes=False)

    @pl.kernel(
        compiler_params=cp,
        out_type=jax.ShapeDtypeStruct((NUM_IDX, EMB), embedding_publisher.dtype),
        mesh=mesh,
        scratch_types=[
            pltpu.SMEM((CHUNK,), jnp.int32),
            pltpu.SemaphoreType.DMA,
        ],
    )
    def gather_kernel(table_hbm, idx_hbm, out_hbm, idx_s, sem):
        cid = lax.axis_index("c")
        base = cid * B_PER_W

        @pl.loop(0, NUM_CHUNKS)
        def _(c):
            cbase = base + c * CHUNK
            pltpu.sync_copy(idx_hbm.at[pl.ds(cbase, CHUNK)], idx_s)

            @pl.loop(0, CHUNK)
            def _(i):
                pltpu.async_copy(
                    table_hbm.at[idx_s[i]], out_hbm.at[cbase + i], sem
                )

        pltpu.make_async_copy(
            table_hbm.at[pl.ds(0, B_PER_W)],
            out_hbm.at[pl.ds(base, B_PER_W)],
            sem,
        ).wait()

    return gather_kernel(embedding_publisher, idx)


# speedup vs baseline: 1.0030x; 1.0030x over previous
import dataclasses

import jax
import jax.numpy as jnp
from jax import lax
from jax.experimental import pallas as pl
from jax.experimental.pallas import tpu as pltpu
from jax.experimental.pallas import tpu_sc as plsc

NUM_IDX = 16384
EMB = 64
NUM_CORES = 2
B_PER_W = NUM_IDX // NUM_CORES
CHUNK = 512
NUM_CHUNKS = B_PER_W // CHUNK


def kernel(publisher_idx, embedding_publisher):
    idx = publisher_idx.astype(jnp.int32)
    mesh = plsc.ScalarSubcoreMesh(axis_name="c", num_cores=NUM_CORES)
    cp = pltpu.CompilerParams()
    if "needs_layout_passes" in pltpu.CompilerParams.__dataclass_fields__:
        cp = dataclasses.replace(cp, needs_layout_passes=False)

    @pl.kernel(
        compiler_params=cp,
        out_type=jax.ShapeDtypeStruct((NUM_IDX, EMB), embedding_publisher.dtype),
        mesh=mesh,
        scratch_types=[
            pltpu.SMEM((2 * CHUNK,), jnp.int32),
            pltpu.SemaphoreType.DMA,
            pltpu.SemaphoreType.DMA,
        ],
    )
    def gather_kernel(table_hbm, idx_hbm, out_hbm, idx_s, sem, sem_idx):
        cid = lax.axis_index("c")
        base = cid * B_PER_W

        pltpu.async_copy(
            idx_hbm.at[pl.ds(base, CHUNK)], idx_s.at[pl.ds(0, CHUNK)], sem_idx
        )

        @pl.loop(0, NUM_CHUNKS)
        def _(c):
            cbase = base + c * CHUNK
            boff = (c % 2) * CHUNK
            pltpu.make_async_copy(
                idx_hbm.at[pl.ds(cbase, CHUNK)],
                idx_s.at[pl.ds(boff, CHUNK)],
                sem_idx,
            ).wait()

            @pl.when(c + 1 < NUM_CHUNKS)
            def _():
                pltpu.async_copy(
                    idx_hbm.at[pl.ds(cbase + CHUNK, CHUNK)],
                    idx_s.at[pl.ds(((c + 1) % 2) * CHUNK, CHUNK)],
                    sem_idx,
                )

            @pl.loop(0, CHUNK, unroll=8)
            def _(i):
                pltpu.async_copy(
                    table_hbm.at[idx_s[boff + i]], out_hbm.at[cbase + i], sem
                )

        pltpu.make_async_copy(
            table_hbm.at[pl.ds(0, B_PER_W)],
            out_hbm.at[pl.ds(base, B_PER_W)],
            sem,
        ).wait()

    return gather_kernel(embedding_publisher, idx)


# speedup vs baseline: 2.3123x; 2.3053x over previous
import dataclasses

import jax
import jax.numpy as jnp
from jax import lax
from jax.experimental import pallas as pl
from jax.experimental.pallas import tpu as pltpu
from jax.experimental.pallas import tpu_sc as plsc

NUM_IDX = 16384
EMB = 64
WIDE = 2 * EMB
NUM_CORES = 2
NUM_SUBCORES = 16
NUM_WORKERS = NUM_CORES * NUM_SUBCORES
B_PER_W = NUM_IDX // NUM_WORKERS
CHUNK = 128
NUM_CHUNKS = B_PER_W // CHUNK


def kernel(publisher_idx, embedding_publisher):
    idx = publisher_idx.astype(jnp.int32)
    wide = jnp.concatenate(
        [embedding_publisher, jnp.roll(embedding_publisher, -1, axis=0)],
        axis=1,
    )
    mesh = plsc.VectorSubcoreMesh(core_axis_name="c", subcore_axis_name="s")
    cp = pltpu.CompilerParams()
    if "needs_layout_passes" in pltpu.CompilerParams.__dataclass_fields__:
        cp = dataclasses.replace(cp, needs_layout_passes=False)

    @pl.kernel(
        compiler_params=cp,
        out_type=jax.ShapeDtypeStruct((NUM_IDX, WIDE), embedding_publisher.dtype),
        mesh=mesh,
        scratch_types=[
            pltpu.VMEM((B_PER_W,), jnp.int32),
            pltpu.VMEM((B_PER_W, WIDE), jnp.float32),
            pltpu.SemaphoreType.DMA,
        ],
    )
    def gather_kernel(table_hbm, idx_hbm, out_hbm, idx_v, rows_v, sem):
        wid = lax.axis_index("s") * NUM_CORES + lax.axis_index("c")
        base = wid * B_PER_W
        pltpu.sync_copy(idx_hbm.at[pl.ds(base, B_PER_W)], idx_v)

        for j in range(NUM_CHUNKS):
            pltpu.async_copy(
                table_hbm.at[idx_v.at[pl.ds(j * CHUNK, CHUNK)]],
                rows_v.at[pl.ds(j * CHUNK, CHUNK)],
                sem,
            )
        pltpu.make_async_copy(
            table_hbm.at[pl.ds(0, B_PER_W)], rows_v, sem
        ).wait()

        pltpu.sync_copy(rows_v, out_hbm.at[pl.ds(base, B_PER_W)])

    return gather_kernel(wide, idx)[:, :EMB]


# speedup vs baseline: 3.6462x; 1.5769x over previous
import dataclasses

import jax
import jax.numpy as jnp
from jax import lax
from jax.experimental import pallas as pl
from jax.experimental.pallas import tpu as pltpu
from jax.experimental.pallas import tpu_sc as plsc

NUM_IDX = 16384
EMB = 64
WIDE = 2 * EMB
NUM_CORES = 2
NUM_SUBCORES = 16
NUM_WORKERS = NUM_CORES * NUM_SUBCORES
B_PER_W = NUM_IDX // NUM_WORKERS
CHUNK = 128
NUM_CHUNKS = B_PER_W // CHUNK


def kernel(publisher_idx, embedding_publisher):
    idx = publisher_idx.astype(jnp.int32)
    wide = jnp.pad(embedding_publisher, ((0, 0), (0, WIDE - EMB)))
    mesh = plsc.VectorSubcoreMesh(core_axis_name="c", subcore_axis_name="s")
    cp = pltpu.CompilerParams()
    if "needs_layout_passes" in pltpu.CompilerParams.__dataclass_fields__:
        cp = dataclasses.replace(cp, needs_layout_passes=False)

    @pl.kernel(
        compiler_params=cp,
        out_type=jax.ShapeDtypeStruct((NUM_IDX, WIDE), embedding_publisher.dtype),
        mesh=mesh,
        scratch_types=[
            pltpu.VMEM((B_PER_W,), jnp.int32),
            pltpu.VMEM((B_PER_W, WIDE), jnp.float32),
            pltpu.SemaphoreType.DMA,
        ],
    )
    def gather_kernel(table_hbm, idx_hbm, out_hbm, idx_v, rows_v, sem):
        wid = lax.axis_index("s") * NUM_CORES + lax.axis_index("c")
        base = wid * B_PER_W
        pltpu.sync_copy(idx_hbm.at[pl.ds(base, B_PER_W)], idx_v)

        for j in range(NUM_CHUNKS):
            pltpu.async_copy(
                table_hbm.at[idx_v.at[pl.ds(j * CHUNK, CHUNK)]],
                rows_v.at[pl.ds(j * CHUNK, CHUNK)],
                sem,
            )
        pltpu.make_async_copy(
            table_hbm.at[pl.ds(0, B_PER_W)], rows_v, sem
        ).wait()

        pltpu.sync_copy(rows_v, out_hbm.at[pl.ds(base, B_PER_W)])

    return gather_kernel(wide, idx)[:, :EMB]
